# baseline (speedup 1.0000x reference)
import functools
import math

import jax
import jax.numpy as jnp
from jax import lax
from jax.experimental import pallas as pl
from jax.experimental.pallas import tpu as pltpu
from jax.experimental.pallas import tpu_sc as plsc

_SCALE = math.sqrt(128.0)
_CHUNK = 128
_NBUF = 4


def _make_gather(vocab, dim, n_idx):
    info = plsc.get_sparse_core_info()
    nc, ns = info.num_cores, info.num_subcores
    nw = nc * ns
    assert n_idx % (nw * _CHUNK) == 0
    per_w = n_idx // nw
    n_chunks = per_w // _CHUNK
    assert n_chunks % _NBUF == 0
    n_groups = n_chunks // _NBUF

    mesh = plsc.VectorSubcoreMesh(core_axis_name="c", subcore_axis_name="s")

    @functools.partial(
        pl.kernel,
        mesh=mesh,
        out_type=jax.ShapeDtypeStruct((n_idx, dim), jnp.float32),
        scratch_types=[
            pltpu.VMEM((n_chunks, _CHUNK), jnp.int32),
            *([pltpu.VMEM((_CHUNK, dim), jnp.float32)] * _NBUF),
            *([pltpu.SemaphoreType.DMA] * (2 * _NBUF)),
        ],
    )
    def gather_k(table_hbm, idx_hbm, out_hbm, idx_v, *bufs_and_sems):
        rows = bufs_and_sems[:_NBUF]
        gsem = bufs_and_sems[_NBUF : 2 * _NBUF]
        osem = bufs_and_sems[2 * _NBUF :]
        wid = lax.axis_index("s") * nc + lax.axis_index("c")
        base = wid * per_w
        pltpu.sync_copy(idx_hbm.at[pl.ds(wid * n_chunks, n_chunks)], idx_v)

        def body(g, carry):
            first = g * _NBUF
            for b in range(_NBUF):
                @pl.when(g > 0)
                def _():
                    pltpu.make_async_copy(
                        rows[b], out_hbm.at[pl.ds(0, _CHUNK)], osem[b]
                    ).wait()
                pltpu.async_copy(
                    table_hbm.at[idx_v.at[first + b]], rows[b], gsem[b]
                )
            for b in range(_NBUF):
                pltpu.make_async_copy(
                    table_hbm.at[idx_v.at[first + b]], rows[b], gsem[b]
                ).wait()

                def sbody(r, c, buf=rows[b]):
                    for j in range(dim // 16):
                        buf[r, pl.ds(j * 16, 16)] = (
                            buf[r, pl.ds(j * 16, 16)] * _SCALE
                        )
                    return c

                lax.fori_loop(0, _CHUNK, sbody, 0)
                off = base + (first + b) * _CHUNK
                pltpu.async_copy(rows[b], out_hbm.at[pl.ds(off, _CHUNK)], osem[b])
            return carry

        lax.fori_loop(0, n_groups, body, 0)
        for b in range(_NBUF):
            pltpu.make_async_copy(
                rows[b], out_hbm.at[pl.ds(0, _CHUNK)], osem[b]
            ).wait()

    return gather_k


def kernel(x, table):
    vocab, dim = table.shape
    x_flat = x.reshape(-1).astype(jnp.int32)
    n_idx = x_flat.shape[0]
    idx2d = x_flat.reshape(-1, _CHUNK)
    out = _make_gather(vocab, dim, n_idx)(table, idx2d)
    return out.reshape(x.shape + (dim,))
